# baseline (speedup 1.0000x reference)
import functools

import jax
import jax.numpy as jnp
from jax import lax
from jax.experimental import pallas as pl
from jax.experimental.pallas import tpu as pltpu, tpu_sc as plsc

N = 10000
E = 320000
NFEAT = 128
NHID = 128
NCLASS = 64

NC = 2
NS = 16
NW = NC * NS
EPW = E // NW
C = 128
E_PAD = NW * 10240
EPW_PAD = E_PAD // NW
NCHUNK = EPW_PAD // C
SEED = 80
N_PAD = 10240
RPT = N_PAD // NS
ROW_BLK = 1280


def _seg_kernel_body(with_deg, *refs):
    if with_deg:
        (h_hbm, src_hbm, dst_hbm, parts_hbm, degp_hbm,
         src_v, dst_v, rows0_v, rows1_v, ones_v, zrow_v,
         agg_sh, deg_sh, sem0, sem1) = refs
    else:
        (h_hbm, src_hbm, dst_hbm, parts_hbm,
         src_v, dst_v, rows0_v, rows1_v, ones_v, zrow_v,
         agg_sh, deg_sh, sem0, sem1) = refs

    cid = lax.axis_index("c")
    sid = lax.axis_index("s")
    wid = cid * NS + sid
    row0 = sid * RPT

    @pl.loop(0, 8)
    def _(i):
        ones_v[pl.ds(i * 16, 16)] = jnp.full((16,), 1.0, jnp.float32)
        zrow_v[pl.ds(i * 16, 16)] = jnp.zeros((16,), jnp.float32)

    @pl.when(cid == 0)
    def _():
        pltpu.sync_copy(h_hbm.at[pl.ds(row0, RPT)], agg_sh.at[pl.ds(row0, RPT)])
        if with_deg:
            @pl.loop(0, RPT // SEED)
            def _(t):
                pltpu.sync_copy(ones_v.at[pl.ds(0, SEED)],
                                deg_sh.at[pl.ds(row0 + t * SEED, SEED)])

    @pl.when(cid != 0)
    def _():
        @pl.loop(0, SEED * NFEAT // 16)
        def _(i):
            r = i // (NFEAT // 16)
            c = i % (NFEAT // 16)
            rows0_v[r, pl.ds(c * 16, 16)] = jnp.zeros((16,), jnp.float32)

        @pl.loop(0, RPT // SEED)
        def _(t):
            pltpu.sync_copy(rows0_v.at[pl.ds(0, SEED)],
                            agg_sh.at[pl.ds(row0 + t * SEED, SEED)])
            if with_deg:
                pltpu.sync_copy(zrow_v.at[pl.ds(0, SEED)],
                                deg_sh.at[pl.ds(row0 + t * SEED, SEED)])

    plsc.subcore_barrier()

    def _scatter(j, buf):
        pltpu.sync_copy(buf, agg_sh.at[dst_v.at[j]], add=True)
        if with_deg:
            pltpu.sync_copy(ones_v, deg_sh.at[dst_v.at[j]], add=True)

    SECT = NCHUNK // 2
    for sect in range(2):
        pltpu.sync_copy(src_hbm.at[wid, pl.ds(sect * SECT, SECT)], src_v)
        pltpu.sync_copy(dst_hbm.at[wid, pl.ds(sect * SECT, SECT)], dst_v)

        @pl.loop(0, SECT, step=2)
        def _(j):
            d0 = pltpu.async_copy(h_hbm.at[src_v.at[j]], rows0_v, sem0)
            d0.wait()
            d1 = pltpu.async_copy(h_hbm.at[src_v.at[j + 1]], rows1_v, sem1)
            _scatter(j, rows0_v)
            d1.wait()
            _scatter(j + 1, rows1_v)

    plsc.subcore_barrier()

    pltpu.sync_copy(agg_sh.at[pl.ds(row0, RPT)],
                    parts_hbm.at[cid, pl.ds(row0, RPT)])
    if with_deg:
        @pl.when(sid == 0)
        def _():
            pltpu.sync_copy(deg_sh, degp_hbm.at[cid])


def _make_seg_kernel(with_deg):
    out_type = [jax.ShapeDtypeStruct((NC, N_PAD, NFEAT), jnp.float32)]
    if with_deg:
        out_type.append(jax.ShapeDtypeStruct((NC, N_PAD), jnp.float32))
    return pl.kernel(
        functools.partial(_seg_kernel_body, with_deg),
        out_type=out_type,
        mesh=plsc.VectorSubcoreMesh(core_axis_name="c", subcore_axis_name="s"),
        scratch_types=[
            pltpu.VMEM((NCHUNK // 2, C), jnp.int32),
            pltpu.VMEM((NCHUNK // 2, C), jnp.int32),
            pltpu.VMEM((C, NFEAT), jnp.float32),
            pltpu.VMEM((C, NFEAT), jnp.float32),
            pltpu.VMEM((128,), jnp.float32),
            pltpu.VMEM((128,), jnp.float32),
            pltpu.VMEM_SHARED((N_PAD, NFEAT), jnp.float32),
            pltpu.VMEM_SHARED((N_PAD,), jnp.float32),
            pltpu.SemaphoreType.DMA,
            pltpu.SemaphoreType.DMA,
        ],
    )


_seg_with_deg = _make_seg_kernel(True)
_seg_no_deg = _make_seg_kernel(False)


def _tc1_body(parts_ref, degp_ref, w_ref, b_ref, out_ref):
    p = parts_ref[0] + parts_ref[1]
    deg = degp_ref[0, 0] + degp_ref[0, 1]
    hm = p / deg[:, None]
    z = jnp.dot(hm, w_ref[...], preferred_element_type=jnp.float32) + b_ref[...]
    nrm = jnp.sqrt(jnp.sum(z * z, axis=1, keepdims=True))
    z = z / jnp.maximum(nrm, 1e-12)
    out_ref[...] = jnp.maximum(z, 0.0)


def _tc2_body(parts_ref, degp_ref, w_ref, b_ref, out_ref):
    p = parts_ref[0] + parts_ref[1]
    deg = degp_ref[0, 0] + degp_ref[0, 1]
    hm = p / deg[:, None]
    z = jnp.dot(hm, w_ref[...], preferred_element_type=jnp.float32) + b_ref[...]
    m = jnp.max(z, axis=1, keepdims=True)
    s = z - m
    lse = jnp.log(jnp.sum(jnp.exp(s), axis=1, keepdims=True))
    out_ref[...] = s - lse


def _tc_layer(body, parts, degp_r, w, b, ncols):
    nblk = N_PAD // ROW_BLK
    return pl.pallas_call(
        body,
        grid=(nblk,),
        in_specs=[
            pl.BlockSpec((NC, ROW_BLK, NFEAT), lambda i: (0, i, 0)),
            pl.BlockSpec((1, NC, ROW_BLK), lambda i: (i, 0, 0)),
            pl.BlockSpec((NFEAT, ncols), lambda i: (0, 0)),
            pl.BlockSpec((1, ncols), lambda i: (0, 0)),
        ],
        out_specs=pl.BlockSpec((ROW_BLK, ncols), lambda i: (i, 0)),
        out_shape=jax.ShapeDtypeStruct((N_PAD, ncols), jnp.float32),
    )(parts, degp_r, w, b)


def kernel(x, edge_index, W0, b0, W1, b1):
    npad = E_PAD - E
    src_pad = jnp.concatenate(
        [edge_index[0], jnp.zeros((npad,), jnp.int32)])
    dst_pad = jnp.concatenate(
        [edge_index[1], jnp.full((npad,), N, jnp.int32)])
    src2d = src_pad.reshape(NW, NCHUNK, C)
    dst2d = dst_pad.reshape(NW, NCHUNK, C)
    x_pad = jnp.pad(x, ((0, N_PAD - N), (0, 0)))

    parts1, degp = _seg_with_deg(x_pad, src2d, dst2d)
    degp_r = degp.reshape(NC, N_PAD // ROW_BLK, ROW_BLK).transpose(1, 0, 2)
    h1 = _tc_layer(_tc1_body, parts1, degp_r, W0, b0, NHID)
    (parts2,) = _seg_no_deg(h1, src2d, dst2d)
    out = _tc_layer(_tc2_body, parts2, degp_r, W1, b1, NCLASS)
    return out[:N]


# speedup vs baseline: 3.1314x; 3.1314x over previous
import functools

import jax
import jax.numpy as jnp
from jax import lax
from jax.experimental import pallas as pl
from jax.experimental.pallas import tpu as pltpu, tpu_sc as plsc

N = 10000
E = 320000
NFEAT = 128
NHID = 128
NCLASS = 64

NC = 2
NS = 16
NW = NC * NS
EPW = E // NW
C = 125
NCHUNK = EPW // C
SEED = 80
N_PAD = 10240
RPT = N_PAD // NS
ROW_BLK = 1280


def _seg_kernel_body(with_deg, *refs):
    if with_deg:
        (h_hbm, src_hbm, dst_hbm, parts_hbm, degp_hbm,
         src_v, dst_v, rows0_v, rows1_v, ones_v, zrow_v,
         agg_sh, deg_sh, sem0, sem1) = refs
    else:
        (h_hbm, src_hbm, dst_hbm, parts_hbm,
         src_v, dst_v, rows0_v, rows1_v, ones_v, zrow_v,
         agg_sh, deg_sh, sem0, sem1) = refs

    cid = lax.axis_index("c")
    sid = lax.axis_index("s")
    wid = cid * NS + sid
    row0 = sid * RPT

    @pl.loop(0, 8)
    def _(i):
        ones_v[pl.ds(i * 16, 16)] = jnp.full((16,), 1.0, jnp.float32)
        zrow_v[pl.ds(i * 16, 16)] = jnp.zeros((16,), jnp.float32)

    @pl.when(cid == 0)
    def _():
        pltpu.sync_copy(h_hbm.at[pl.ds(row0, RPT)], agg_sh.at[pl.ds(row0, RPT)])
        if with_deg:
            @pl.loop(0, RPT // SEED)
            def _(t):
                pltpu.sync_copy(ones_v.at[pl.ds(0, SEED)],
                                deg_sh.at[pl.ds(row0 + t * SEED, SEED)])

    @pl.when(cid != 0)
    def _():
        @pl.loop(0, SEED * NFEAT // 16)
        def _(i):
            r = i // (NFEAT // 16)
            c = i % (NFEAT // 16)
            rows0_v[r, pl.ds(c * 16, 16)] = jnp.zeros((16,), jnp.float32)

        @pl.loop(0, RPT // SEED)
        def _(t):
            pltpu.sync_copy(rows0_v.at[pl.ds(0, SEED)],
                            agg_sh.at[pl.ds(row0 + t * SEED, SEED)])
            if with_deg:
                pltpu.sync_copy(zrow_v.at[pl.ds(0, SEED)],
                                deg_sh.at[pl.ds(row0 + t * SEED, SEED)])

    plsc.subcore_barrier()

    def _scatter(j, buf):
        pltpu.sync_copy(buf, agg_sh.at[dst_v.at[j]], add=True)
        if with_deg:
            pltpu.sync_copy(ones_v.at[pl.ds(0, C)],
                            deg_sh.at[dst_v.at[j]], add=True)

    SECT = NCHUNK // 2
    for sect in range(2):
        pltpu.sync_copy(src_hbm.at[wid, pl.ds(sect * SECT, SECT)], src_v)
        pltpu.sync_copy(dst_hbm.at[wid, pl.ds(sect * SECT, SECT)], dst_v)

        @pl.loop(0, SECT, step=2)
        def _(j):
            d0 = pltpu.async_copy(h_hbm.at[src_v.at[j]], rows0_v, sem0)
            d0.wait()
            d1 = pltpu.async_copy(h_hbm.at[src_v.at[j + 1]], rows1_v, sem1)
            _scatter(j, rows0_v)
            d1.wait()
            _scatter(j + 1, rows1_v)

    plsc.subcore_barrier()

    pltpu.sync_copy(agg_sh.at[pl.ds(row0, RPT)],
                    parts_hbm.at[cid, pl.ds(row0, RPT)])
    if with_deg:
        @pl.when(sid == 0)
        def _():
            pltpu.sync_copy(deg_sh, degp_hbm.at[cid])


def _make_seg_kernel(with_deg):
    out_type = [jax.ShapeDtypeStruct((NC, N_PAD, NFEAT), jnp.float32)]
    if with_deg:
        out_type.append(jax.ShapeDtypeStruct((NC, N_PAD), jnp.float32))
    return pl.kernel(
        functools.partial(_seg_kernel_body, with_deg),
        out_type=out_type,
        mesh=plsc.VectorSubcoreMesh(core_axis_name="c", subcore_axis_name="s"),
        scratch_types=[
            pltpu.VMEM((NCHUNK // 2, C), jnp.int32),
            pltpu.VMEM((NCHUNK // 2, C), jnp.int32),
            pltpu.VMEM((C, NFEAT), jnp.float32),
            pltpu.VMEM((C, NFEAT), jnp.float32),
            pltpu.VMEM((128,), jnp.float32),
            pltpu.VMEM((128,), jnp.float32),
            pltpu.VMEM_SHARED((N_PAD, NFEAT), jnp.float32),
            pltpu.VMEM_SHARED((N_PAD,), jnp.float32),
            pltpu.SemaphoreType.DMA,
            pltpu.SemaphoreType.DMA,
        ],
    )


_seg_with_deg = _make_seg_kernel(True)
_seg_no_deg = _make_seg_kernel(False)


def _tc1_body(parts_ref, degp_ref, w_ref, b_ref, out_ref):
    p = parts_ref[0] + parts_ref[1]
    deg = degp_ref[0, 0] + degp_ref[0, 1]
    hm = p / deg[:, None]
    z = jnp.dot(hm, w_ref[...], preferred_element_type=jnp.float32) + b_ref[...]
    nrm = jnp.sqrt(jnp.sum(z * z, axis=1, keepdims=True))
    z = z / jnp.maximum(nrm, 1e-12)
    out_ref[...] = jnp.maximum(z, 0.0)


def _tc2_body(parts_ref, degp_ref, w_ref, b_ref, out_ref):
    p = parts_ref[0] + parts_ref[1]
    deg = degp_ref[0, 0] + degp_ref[0, 1]
    hm = p / deg[:, None]
    z = jnp.dot(hm, w_ref[...], preferred_element_type=jnp.float32) + b_ref[...]
    m = jnp.max(z, axis=1, keepdims=True)
    s = z - m
    lse = jnp.log(jnp.sum(jnp.exp(s), axis=1, keepdims=True))
    out_ref[...] = s - lse


def _tc_layer(body, parts, degp_r, w, b, ncols):
    nblk = N_PAD // ROW_BLK
    return pl.pallas_call(
        body,
        grid=(nblk,),
        in_specs=[
            pl.BlockSpec((NC, ROW_BLK, NFEAT), lambda i: (0, i, 0)),
            pl.BlockSpec((1, NC, ROW_BLK), lambda i: (i, 0, 0)),
            pl.BlockSpec((NFEAT, ncols), lambda i: (0, 0)),
            pl.BlockSpec((1, ncols), lambda i: (0, 0)),
        ],
        out_specs=pl.BlockSpec((ROW_BLK, ncols), lambda i: (i, 0)),
        out_shape=jax.ShapeDtypeStruct((N_PAD, ncols), jnp.float32),
    )(parts, degp_r, w, b)


def kernel(x, edge_index, W0, b0, W1, b1):
    src2d = edge_index[0].reshape(NW, NCHUNK, C)
    dst2d = edge_index[1].reshape(NW, NCHUNK, C)
    x_pad = jnp.pad(x, ((0, N_PAD - N), (0, 0)))

    parts1, degp = _seg_with_deg(x_pad, src2d, dst2d)
    degp_r = degp.reshape(NC, N_PAD // ROW_BLK, ROW_BLK).transpose(1, 0, 2)
    h1 = _tc_layer(_tc1_body, parts1, degp_r, W0, b0, NHID)
    (parts2,) = _seg_no_deg(h1, src2d, dst2d)
    out = _tc_layer(_tc2_body, parts2, degp_r, W1, b1, NCLASS)
    return out[:N]


# speedup vs baseline: 4.0631x; 1.2976x over previous
import functools

import jax
import jax.numpy as jnp
from jax import lax
from jax.experimental import pallas as pl
from jax.experimental.pallas import tpu as pltpu, tpu_sc as plsc

N = 10000
E = 320000
NFEAT = 128
NHID = 128
NCLASS = 64

NC = 2
NS = 16
NW = NC * NS
EPW = E // NW
C = 125
NCHUNK = EPW // C
SEED = 80
N_PAD = 10240
RPT = N_PAD // NS
ROW_BLK = 1280


def _seg_kernel_body(with_deg, *refs):
    if with_deg:
        (h_hbm, src_hbm, dst_hbm, parts_hbm, degp_hbm,
         src_v, dst_v, rows0_v, rows1_v, ones_v, zrow_v,
         agg_sh, deg_sh, sem0, sem1) = refs
    else:
        (h_hbm, src_hbm, dst_hbm, parts_hbm,
         src_v, dst_v, rows0_v, rows1_v, ones_v, zrow_v,
         agg_sh, deg_sh, sem0, sem1) = refs

    cid = lax.axis_index("c")
    sid = lax.axis_index("s")
    wid = cid * NS + sid
    row0 = sid * RPT

    @pl.loop(0, 8)
    def _(i):
        ones_v[pl.ds(i * 16, 16)] = jnp.full((16,), 1.0, jnp.float32)
        zrow_v[pl.ds(i * 16, 16)] = jnp.zeros((16,), jnp.float32)

    @pl.when(cid == 0)
    def _():
        pltpu.sync_copy(h_hbm.at[pl.ds(row0, RPT)], agg_sh.at[pl.ds(row0, RPT)])
        if with_deg:
            @pl.loop(0, RPT // SEED)
            def _(t):
                pltpu.sync_copy(ones_v.at[pl.ds(0, SEED)],
                                deg_sh.at[pl.ds(row0 + t * SEED, SEED)])

    @pl.when(cid != 0)
    def _():
        @pl.loop(0, SEED * NFEAT // 16)
        def _(i):
            r = i // (NFEAT // 16)
            c = i % (NFEAT // 16)
            rows0_v[r, pl.ds(c * 16, 16)] = jnp.zeros((16,), jnp.float32)

        @pl.loop(0, RPT // SEED)
        def _(t):
            pltpu.sync_copy(rows0_v.at[pl.ds(0, SEED)],
                            agg_sh.at[pl.ds(row0 + t * SEED, SEED)])
            if with_deg:
                pltpu.sync_copy(zrow_v.at[pl.ds(0, SEED)],
                                deg_sh.at[pl.ds(row0 + t * SEED, SEED)])

    plsc.subcore_barrier()

    def _scatter(j, buf):
        pltpu.sync_copy(buf, agg_sh.at[dst_v.at[j]], add=True)
        if with_deg:
            pltpu.sync_copy(ones_v.at[pl.ds(0, C)],
                            deg_sh.at[dst_v.at[j]], add=True)

    def _start(j, buf, sem):
        pltpu.async_copy(h_hbm.at[src_v.at[j]], buf, sem)

    def _wait(j, buf, sem):
        pltpu.make_async_copy(h_hbm.at[src_v.at[j]], buf, sem).wait()

    SECT = NCHUNK // 2
    for sect in range(2):
        pltpu.sync_copy(src_hbm.at[wid, pl.ds(sect * SECT, SECT)], src_v)
        pltpu.sync_copy(dst_hbm.at[wid, pl.ds(sect * SECT, SECT)], dst_v)

        _start(0, rows0_v, sem0)

        @pl.loop(0, SECT, step=2)
        def _(j):
            _start(j + 1, rows1_v, sem1)
            _wait(j, rows0_v, sem0)
            _scatter(j, rows0_v)

            @pl.when(j + 2 < SECT)
            def _():
                _start(j + 2, rows0_v, sem0)

            _wait(j + 1, rows1_v, sem1)
            _scatter(j + 1, rows1_v)

    plsc.subcore_barrier()

    pltpu.sync_copy(agg_sh.at[pl.ds(row0, RPT)],
                    parts_hbm.at[cid, pl.ds(row0, RPT)])
    if with_deg:
        @pl.when(sid == 0)
        def _():
            pltpu.sync_copy(deg_sh, degp_hbm.at[cid])


def _make_seg_kernel(with_deg):
    out_type = [jax.ShapeDtypeStruct((NC, N_PAD, NFEAT), jnp.float32)]
    if with_deg:
        out_type.append(jax.ShapeDtypeStruct((NC, N_PAD), jnp.float32))
    return pl.kernel(
        functools.partial(_seg_kernel_body, with_deg),
        out_type=out_type,
        mesh=plsc.VectorSubcoreMesh(core_axis_name="c", subcore_axis_name="s"),
        scratch_types=[
            pltpu.VMEM((NCHUNK // 2, C), jnp.int32),
            pltpu.VMEM((NCHUNK // 2, C), jnp.int32),
            pltpu.VMEM((C, NFEAT), jnp.float32),
            pltpu.VMEM((C, NFEAT), jnp.float32),
            pltpu.VMEM((128,), jnp.float32),
            pltpu.VMEM((128,), jnp.float32),
            pltpu.VMEM_SHARED((N_PAD, NFEAT), jnp.float32),
            pltpu.VMEM_SHARED((N_PAD,), jnp.float32),
            pltpu.SemaphoreType.DMA,
            pltpu.SemaphoreType.DMA,
        ],
    )


_seg_with_deg = _make_seg_kernel(True)
_seg_no_deg = _make_seg_kernel(False)


def _tc1_body(parts_ref, degp_ref, w_ref, b_ref, out_ref):
    p = parts_ref[0] + parts_ref[1]
    deg = degp_ref[0, 0] + degp_ref[0, 1]
    hm = p / deg[:, None]
    z = jnp.dot(hm, w_ref[...], preferred_element_type=jnp.float32) + b_ref[...]
    nrm = jnp.sqrt(jnp.sum(z * z, axis=1, keepdims=True))
    z = z / jnp.maximum(nrm, 1e-12)
    out_ref[...] = jnp.maximum(z, 0.0)


def _tc2_body(parts_ref, degp_ref, w_ref, b_ref, out_ref):
    p = parts_ref[0] + parts_ref[1]
    deg = degp_ref[0, 0] + degp_ref[0, 1]
    hm = p / deg[:, None]
    z = jnp.dot(hm, w_ref[...], preferred_element_type=jnp.float32) + b_ref[...]
    m = jnp.max(z, axis=1, keepdims=True)
    s = z - m
    lse = jnp.log(jnp.sum(jnp.exp(s), axis=1, keepdims=True))
    out_ref[...] = s - lse


def _tc_layer(body, parts, degp_r, w, b, ncols):
    nblk = N_PAD // ROW_BLK
    return pl.pallas_call(
        body,
        grid=(nblk,),
        in_specs=[
            pl.BlockSpec((NC, ROW_BLK, NFEAT), lambda i: (0, i, 0)),
            pl.BlockSpec((1, NC, ROW_BLK), lambda i: (i, 0, 0)),
            pl.BlockSpec((NFEAT, ncols), lambda i: (0, 0)),
            pl.BlockSpec((1, ncols), lambda i: (0, 0)),
        ],
        out_specs=pl.BlockSpec((ROW_BLK, ncols), lambda i: (i, 0)),
        out_shape=jax.ShapeDtypeStruct((N_PAD, ncols), jnp.float32),
    )(parts, degp_r, w, b)


def kernel(x, edge_index, W0, b0, W1, b1):
    src2d = edge_index[0].reshape(NW, NCHUNK, C)
    dst2d = edge_index[1].reshape(NW, NCHUNK, C)
    x_pad = jnp.pad(x, ((0, N_PAD - N), (0, 0)))

    parts1, degp = _seg_with_deg(x_pad, src2d, dst2d)
    degp_r = degp.reshape(NC, N_PAD // ROW_BLK, ROW_BLK).transpose(1, 0, 2)
    h1 = _tc_layer(_tc1_body, parts1, degp_r, W0, b0, NHID)
    (parts2,) = _seg_no_deg(h1, src2d, dst2d)
    out = _tc_layer(_tc2_body, parts2, degp_r, W1, b1, NCLASS)
    return out[:N]
